# baseline (speedup 1.0000x reference)
import jax
import jax.numpy as jnp
from jax.experimental import pallas as pl
from jax.experimental.pallas import tpu as pltpu

_RADIUS = 1.0
_MAX_K = 64
_MIN_D = 20.0
_MAX_D = 60.0


def _mlp_kernel(w2_ref, w1_ref, b1_ref, b2_ref, w3_ref, b3_smem, out_ref):
    vol = 4.0 / 3.0 * 3.14159 * _RADIUS**3
    d_mean = jnp.float32(_MAX_K) / jnp.float32(vol)

    h1 = jnp.maximum(d_mean * w1_ref[...] + b1_ref[...], 0.0)
    h2 = jax.lax.dot_general(
        h1, w2_ref[...], (((1,), (1,)), ((), ())),
        preferred_element_type=jnp.float32)
    h2 = jnp.maximum(h2 + b2_ref[...], 0.0)
    z = jnp.sum(h2 * w3_ref[...], axis=-1, keepdims=True) + b3_smem[0]
    t = jax.nn.sigmoid(z)
    thr = _MIN_D + (_MAX_D - _MIN_D) * t
    out_ref[...] = jnp.broadcast_to(thr, out_ref.shape)


def kernel(xyz, W1, b1, W2, b2, W3, b3):
    B = xyz.shape[0]
    out = pl.pallas_call(
        _mlp_kernel,
        out_shape=jax.ShapeDtypeStruct((1, B), jnp.float32),
        in_specs=[pl.BlockSpec(memory_space=pltpu.MemorySpace.VMEM)] * 5
        + [pl.BlockSpec(memory_space=pltpu.SMEM)],
    )(
        W2,
        W1.reshape(1, -1),
        b1.reshape(1, -1),
        b2.reshape(1, -1),
        W3.reshape(1, -1),
        b3,
    )
    return out.reshape(B)
